# baseline (speedup 1.0000x reference)
import jax
import jax.numpy as jnp
from jax import lax
from jax.experimental import pallas as pl
from jax.experimental.pallas import tpu as pltpu
from jax.experimental.pallas import tpu_sc as plsc

N = 50000
E = 800000
D_IN = 128
H = 64
OUT = 64
G = 64

CHUNK = 128
HW = 32
NCHUNK = 6272
CPS = NCHUNK // 16
CPS_DEG = NCHUNK // 32
SS = 28
NSUP = CPS // SS
E_PAD = NCHUNK * CHUNK
DUMMY = N
ACC_ROWS = 51200
N_PAD = 50048
WB = N_PAD // 16
ZB = ACC_ROWS // 16 // CHUNK

BM = 2000
GRID = N // BM

_MESH = plsc.VectorSubcoreMesh(core_axis_name="c", subcore_axis_name="s")
_SC_PARAMS = pltpu.CompilerParams(use_tc_tiling_on_sc=False,
                                  needs_layout_passes=False)
_PREC = lax.Precision.HIGHEST


def _agg_body(flag_hbm, y2_hbm, src_hbm, dst_hbm, out_hbm, deg_hbm,
              sidx, didx, rows0, rows1, zbuf, fbuf, acc, sem0, sem1, semi):
    c = lax.axis_index("c")
    s = lax.axis_index("s")
    one32 = jnp.full((32,), 1.0, jnp.bfloat16)
    zero32 = jnp.zeros((32,), jnp.bfloat16)

    @pl.loop(0, CHUNK)
    def _(r):
        rows0[r, pl.ds(0, 32)] = one32
        zbuf[r, pl.ds(0, 32)] = zero32

    pltpu.sync_copy(flag_hbm, fbuf)
    fl = jnp.sum(fbuf[...])

    def zero_acc():
        @pl.loop(0, ZB)
        def _(i):
            pltpu.sync_copy(zbuf, acc.at[pl.ds(s * (ACC_ROWS // 16) + i * CHUNK,
                                               CHUNK)])

    @pl.when(fl > 0)
    def _():
        zero_acc()
        plsc.subcore_barrier()
        cbase = c * (NCHUNK // 2) + s * CPS_DEG

        @pl.loop(0, CPS_DEG // SS)
        def _(sg):
            pltpu.sync_copy(dst_hbm.at[pl.ds(cbase + sg * SS, SS)], didx.at[0])

            @pl.loop(0, SS)
            def _(k):
                pltpu.sync_copy(rows0, acc.at[didx.at[0].at[k]], add=True)

        plsc.subcore_barrier()
        pltpu.sync_copy(acc.at[pl.ds(s * WB, WB)],
                        deg_hbm.at[c].at[pl.ds(s * WB, WB)])
        plsc.subcore_barrier()

    zero_acc()

    def load_super(b, sg):
        base = s * CPS + sg * SS
        pltpu.async_copy(src_hbm.at[pl.ds(base, SS)], sidx.at[b], semi)
        pltpu.async_copy(dst_hbm.at[pl.ds(base, SS)], didx.at[b], semi)

    def wait_super(b, sg):
        base = s * CPS + sg * SS
        pltpu.make_async_copy(src_hbm.at[pl.ds(base, SS)], sidx.at[b],
                              semi).wait()
        pltpu.make_async_copy(dst_hbm.at[pl.ds(base, SS)], didx.at[b],
                              semi).wait()

    def gather_super(b):
        def start(k, rbuf, sem):
            pltpu.async_copy(y2_hbm.at[c].at[sidx.at[b].at[k]], rbuf, sem)

        def finish(k, rbuf, sem):
            pltpu.make_async_copy(y2_hbm.at[c].at[sidx.at[b].at[k]], rbuf,
                                  sem).wait()
            pltpu.sync_copy(rbuf, acc.at[didx.at[b].at[k]], add=True)

        start(0, rows0, sem0)

        @pl.loop(0, SS // 2)
        def _(p):
            k = 2 * p
            start(k + 1, rows1, sem1)
            finish(k, rows0, sem0)

            @pl.when(p < SS // 2 - 1)
            def _():
                start(k + 2, rows0, sem0)

            finish(k + 1, rows1, sem1)

    load_super(0, 0)
    plsc.subcore_barrier()

    @pl.loop(0, NSUP // 2)
    def _(sp):
        sg0 = 2 * sp
        wait_super(0, sg0)
        load_super(1, sg0 + 1)
        gather_super(0)
        wait_super(1, sg0 + 1)

        @pl.when(sp < NSUP // 2 - 1)
        def _():
            load_super(0, sg0 + 2)

        gather_super(1)

    plsc.subcore_barrier()
    pltpu.sync_copy(acc.at[pl.ds(s * WB, WB)],
                    out_hbm.at[c].at[pl.ds(s * WB, WB)])


def _sc_agg(flag, y4, src_p, dst_p):
    k = pl.kernel(
        _agg_body,
        out_type=[jax.ShapeDtypeStruct((2, N_PAD, HW), jnp.bfloat16),
                  jax.ShapeDtypeStruct((2, N_PAD, HW), jnp.bfloat16)],
        mesh=_MESH,
        compiler_params=_SC_PARAMS,
        scratch_types=[
            pltpu.VMEM((2, SS, CHUNK), jnp.int32),
            pltpu.VMEM((2, SS, CHUNK), jnp.int32),
            pltpu.VMEM((CHUNK, HW), jnp.bfloat16),
            pltpu.VMEM((CHUNK, HW), jnp.bfloat16),
            pltpu.VMEM((CHUNK, HW), jnp.bfloat16),
            pltpu.VMEM((16,), jnp.int32),
            pltpu.VMEM_SHARED((ACC_ROWS, HW), jnp.bfloat16),
            pltpu.SemaphoreType.DMA,
            pltpu.SemaphoreType.DMA,
            pltpu.SemaphoreType.DMA,
        ],
    )
    return k(flag, y4, src_p, dst_p)


def _tc_in_body(x_ref, wfc_ref, bfc_ref, wl_ref, wr_ref, b_ref, y4_ref, z_ref):
    h = jnp.dot(x_ref[...], wfc_ref[...], precision=_PREC) + bfc_ref[...]
    h = jnp.maximum(h, 0.0)
    y = jnp.dot(h, wl_ref[...], precision=_PREC).astype(jnp.bfloat16)
    y4_ref[0] = y[:, :HW]
    y4_ref[1] = y[:, HW:]
    z_ref[...] = jnp.dot(h, wr_ref[...], precision=_PREC) + b_ref[...]


def _tc_in(x, W_fc, b_fc, Wl, Wr, b):
    return pl.pallas_call(
        _tc_in_body,
        grid=(GRID,),
        in_specs=[
            pl.BlockSpec((BM, D_IN), lambda i: (i, 0)),
            pl.BlockSpec((D_IN, H), lambda i: (0, 0)),
            pl.BlockSpec((1, H), lambda i: (0, 0)),
            pl.BlockSpec((H, H), lambda i: (0, 0)),
            pl.BlockSpec((H, H), lambda i: (0, 0)),
            pl.BlockSpec((1, H), lambda i: (0, 0)),
        ],
        out_specs=[
            pl.BlockSpec((2, BM, HW), lambda i: (0, i, 0)),
            pl.BlockSpec((BM, H), lambda i: (i, 0)),
        ],
        out_shape=[
            jax.ShapeDtypeStruct((2, N, HW), jnp.bfloat16),
            jax.ShapeDtypeStruct((N, H), jnp.float32),
        ],
    )(x, W_fc, b_fc, Wl, Wr, b)


def _mid_body(l_ref, agg_ref, deg_in_ref, z_ref, deg_ref, g_ref, be_ref,
              wl_ref, wr_ref, b_ref, y4_ref, z2_ref, deg2_ref, traw_ref):
    lv = l_ref[0, 0]
    agg = jnp.concatenate(
        [agg_ref[0], agg_ref[1]], axis=1).astype(jnp.float32)
    deg_new = (deg_in_ref[0, :, 0:1] + deg_in_ref[1, :, 0:1]
               ).astype(jnp.float32)
    deg = jnp.where(lv == 1, deg_new, deg_ref[...])
    deg2_ref[...] = deg
    inv = 1.0 / jnp.maximum(deg, 1.0)
    traw = agg * inv + z_ref[...]
    traw_ref[...] = traw
    mu = jnp.mean(traw, axis=-1, keepdims=True)
    var = jnp.mean((traw - mu) * (traw - mu), axis=-1, keepdims=True)
    t = (traw - mu) * lax.rsqrt(var + 1e-5) * g_ref[...] + be_ref[...]
    t = jnp.maximum(t, 0.0)
    y = jnp.dot(t, wl_ref[...], precision=_PREC).astype(jnp.bfloat16)
    y4_ref[0] = y[:, :HW]
    y4_ref[1] = y[:, HW:]
    z2_ref[...] = jnp.dot(t, wr_ref[...], precision=_PREC) + b_ref[...]


def _tc_mid(lflag, aggM, degM, z, deg, g, be, Wl, Wr, b):
    return pl.pallas_call(
        _mid_body,
        grid=(GRID,),
        in_specs=[
            pl.BlockSpec((1, 1), lambda i: (0, 0)),
            pl.BlockSpec((2, BM, HW), lambda i: (0, i, 0)),
            pl.BlockSpec((2, BM, HW), lambda i: (0, i, 0)),
            pl.BlockSpec((BM, H), lambda i: (i, 0)),
            pl.BlockSpec((BM, 1), lambda i: (i, 0)),
            pl.BlockSpec((1, H), lambda i: (0, 0)),
            pl.BlockSpec((1, H), lambda i: (0, 0)),
            pl.BlockSpec((H, H), lambda i: (0, 0)),
            pl.BlockSpec((H, H), lambda i: (0, 0)),
            pl.BlockSpec((1, H), lambda i: (0, 0)),
        ],
        out_specs=[
            pl.BlockSpec((2, BM, HW), lambda i: (0, i, 0)),
            pl.BlockSpec((BM, H), lambda i: (i, 0)),
            pl.BlockSpec((BM, 1), lambda i: (i, 0)),
            pl.BlockSpec((BM, H), lambda i: (i, 0)),
        ],
        out_shape=[
            jax.ShapeDtypeStruct((2, N, HW), jnp.bfloat16),
            jax.ShapeDtypeStruct((N, H), jnp.float32),
            jax.ShapeDtypeStruct((N, 1), jnp.float32),
            jax.ShapeDtypeStruct((N, H), jnp.float32),
        ],
    )(lflag, aggM, degM, z, deg, g, be, Wl, Wr, b)


def _pool_body(ne_ref, batch_ref, ge_ref):
    i = pl.program_id(0)
    node = ne_ref[...]
    b = batch_ref[0, 0]
    p = (b[:, None] == lax.broadcasted_iota(jnp.int32, (BM, G), 1)
         ).astype(jnp.float32)
    contrib = lax.dot_general(p, node, (((0,), (0,)), ((), ())),
                              precision=_PREC)

    @pl.when(i == 0)
    def _():
        ge_ref[...] = jnp.zeros_like(ge_ref)

    ge_ref[...] += contrib


def _tc_pool(node_embed, batch2):
    return pl.pallas_call(
        _pool_body,
        grid=(GRID,),
        in_specs=[
            pl.BlockSpec((BM, OUT), lambda i: (i, 0)),
            pl.BlockSpec((1, 1, BM), lambda i: (i, 0, 0)),
        ],
        out_specs=pl.BlockSpec((G, OUT), lambda i: (0, 0)),
        out_shape=jax.ShapeDtypeStruct((G, OUT), jnp.float32),
    )(node_embed, batch2)


def kernel(x, edge_index, batch, W_fc, b_fc,
           Wl1, bl1, Wr1, br1, g1, be1,
           Wl2, bl2, Wr2, br2, g2, be2,
           Wl3, bl3, Wr3, br3):
    src = edge_index[0]
    dst = edge_index[1]
    src_p = jnp.concatenate(
        [src, jnp.zeros((E_PAD - E,), jnp.int32)]).reshape(NCHUNK, CHUNK)
    dst_p = jnp.concatenate(
        [dst, jnp.full((E_PAD - E,), DUMMY, jnp.int32)]).reshape(NCHUNK, CHUNK)
    batch2 = batch.reshape(GRID, 1, BM)

    y1, z1 = _tc_in(x, W_fc, b_fc.reshape(1, H), Wl1, Wr1,
                    (bl1 + br1).reshape(1, H))

    ones64 = jnp.ones((1, H), jnp.float32)
    zeros64 = jnp.zeros((1, H), jnp.float32)
    zerosW = jnp.zeros((H, H), jnp.float32)
    xs = (
        jnp.array([[[1]], [[2]], [[3]]], jnp.int32),
        jnp.stack([jnp.ones((16,), jnp.int32)] +
                  [jnp.zeros((16,), jnp.int32)] * 2),
        jnp.stack([g1.reshape(1, H), g2.reshape(1, H), ones64]),
        jnp.stack([be1.reshape(1, H), be2.reshape(1, H), zeros64]),
        jnp.stack([Wl2, Wl3, zerosW]),
        jnp.stack([Wr2, Wr3, zerosW]),
        jnp.stack([(bl2 + br2).reshape(1, H),
                   (bl3 + br3).reshape(1, H), zeros64]),
    )

    def body(carry, xts):
        y4, z, deg, _ = carry
        lflag, scflag, g, be, Wl, Wr, b = xts
        aggM, degM = _sc_agg(scflag, y4, src_p, dst_p)
        y4n, zn, degn, traw = _tc_mid(lflag, aggM, degM, z, deg, g, be,
                                      Wl, Wr, b)
        return (y4n, zn, degn, traw), None

    init = (y1, z1, jnp.zeros((N, 1), jnp.float32),
            jnp.zeros((N, H), jnp.float32))
    (_, _, _, node_embed), _ = lax.scan(body, init, xs)
    graph_embed = _tc_pool(node_embed, batch2)
    return node_embed, graph_embed


# speedup vs baseline: 1.2680x; 1.2680x over previous
import jax
import jax.numpy as jnp
from jax import lax
from jax.experimental import pallas as pl
from jax.experimental.pallas import tpu as pltpu
from jax.experimental.pallas import tpu_sc as plsc

N = 50000
E = 800000
D_IN = 128
H = 64
OUT = 64
G = 64

CHUNK = 128
HW = 32
NCHUNK = 6272
CPS = NCHUNK // 16
CPS_DEG = NCHUNK // 32
SS = 28
NSUP = CPS // SS
E_PAD = NCHUNK * CHUNK
DUMMY = N
ACC_ROWS = 51200
N_PAD = 50048
WB = N_PAD // 16
ZB = ACC_ROWS // 16 // CHUNK

BM = 2000
GRID = N // BM

_SC_PARAMS = pltpu.CompilerParams(use_tc_tiling_on_sc=False,
                                  needs_layout_passes=False)
_PREC = lax.Precision.DEFAULT


def _agg_body(flag_hbm, y2_hbm, src_hbm, dst_hbm, out_hbm, deg_hbm,
              sidx, didx, rows0, rows1, zbuf, fbuf, acc, sem0, sem1, semi):
    c = lax.axis_index("c")
    s = lax.axis_index("s")
    one32 = jnp.full((32,), 1.0, jnp.bfloat16)
    zero32 = jnp.zeros((32,), jnp.bfloat16)

    @pl.loop(0, CHUNK)
    def _(r):
        rows0[r, pl.ds(0, 32)] = one32
        zbuf[r, pl.ds(0, 32)] = zero32

    pltpu.sync_copy(flag_hbm, fbuf)
    fl = jnp.sum(fbuf[...])

    def zero_acc():
        @pl.loop(0, ZB)
        def _(i):
            pltpu.sync_copy(zbuf, acc.at[pl.ds(s * (ACC_ROWS // 16) + i * CHUNK,
                                               CHUNK)])

    @pl.when(fl > 0)
    def _():
        zero_acc()
        plsc.subcore_barrier()
        cbase = c * (NCHUNK // 2) + s * CPS_DEG

        @pl.loop(0, CPS_DEG // SS)
        def _(sg):
            pltpu.sync_copy(dst_hbm.at[pl.ds(cbase + sg * SS, SS)], didx.at[0])

            @pl.loop(0, SS)
            def _(k):
                pltpu.sync_copy(rows0, acc.at[didx.at[0].at[k]], add=True)

        plsc.subcore_barrier()
        pltpu.sync_copy(acc.at[pl.ds(s * WB, WB)],
                        deg_hbm.at[c].at[pl.ds(s * WB, WB)])
        plsc.subcore_barrier()

    zero_acc()

    def load_super(b, sg):
        base = s * CPS + sg * SS
        pltpu.async_copy(src_hbm.at[pl.ds(base, SS)], sidx.at[b], semi)
        pltpu.async_copy(dst_hbm.at[pl.ds(base, SS)], didx.at[b], semi)

    def wait_super(b, sg):
        base = s * CPS + sg * SS
        pltpu.make_async_copy(src_hbm.at[pl.ds(base, SS)], sidx.at[b],
                              semi).wait()
        pltpu.make_async_copy(dst_hbm.at[pl.ds(base, SS)], didx.at[b],
                              semi).wait()

    def gather_super(b):
        def start(k, rbuf, sem):
            pltpu.async_copy(y2_hbm.at[c].at[sidx.at[b].at[k]], rbuf, sem)

        def finish(k, rbuf, sem):
            pltpu.make_async_copy(y2_hbm.at[c].at[sidx.at[b].at[k]], rbuf,
                                  sem).wait()
            pltpu.sync_copy(rbuf, acc.at[didx.at[b].at[k]], add=True)

        start(0, rows0, sem0)

        @pl.loop(0, SS // 2)
        def _(p):
            k = 2 * p
            start(k + 1, rows1, sem1)
            finish(k, rows0, sem0)

            @pl.when(p < SS // 2 - 1)
            def _():
                start(k + 2, rows0, sem0)

            finish(k + 1, rows1, sem1)

    load_super(0, 0)
    plsc.subcore_barrier()

    @pl.loop(0, NSUP // 2)
    def _(sp):
        sg0 = 2 * sp
        wait_super(0, sg0)
        load_super(1, sg0 + 1)
        gather_super(0)
        wait_super(1, sg0 + 1)

        @pl.when(sp < NSUP // 2 - 1)
        def _():
            load_super(0, sg0 + 2)

        gather_super(1)

    plsc.subcore_barrier()
    pltpu.sync_copy(acc.at[pl.ds(s * WB, WB)],
                    out_hbm.at[c].at[pl.ds(s * WB, WB)])


def _sc_agg(flag, y4, src_p, dst_p):
    k = pl.kernel(
        _agg_body,
        out_type=[jax.ShapeDtypeStruct((2, N_PAD, HW), jnp.bfloat16),
                  jax.ShapeDtypeStruct((2, N_PAD, HW), jnp.bfloat16)],
        mesh=plsc.VectorSubcoreMesh(core_axis_name="c", subcore_axis_name="s"),
        compiler_params=_SC_PARAMS,
        scratch_types=[
            pltpu.VMEM((2, SS, CHUNK), jnp.int32),
            pltpu.VMEM((2, SS, CHUNK), jnp.int32),
            pltpu.VMEM((CHUNK, HW), jnp.bfloat16),
            pltpu.VMEM((CHUNK, HW), jnp.bfloat16),
            pltpu.VMEM((CHUNK, HW), jnp.bfloat16),
            pltpu.VMEM((16,), jnp.int32),
            pltpu.VMEM_SHARED((ACC_ROWS, HW), jnp.bfloat16),
            pltpu.SemaphoreType.DMA,
            pltpu.SemaphoreType.DMA,
            pltpu.SemaphoreType.DMA,
        ],
    )
    return k(flag, y4, src_p, dst_p)


def _tc_in_body(x_ref, wfc_ref, bfc_ref, wl_ref, wr_ref, b_ref, y4_ref, z_ref):
    h = jnp.dot(x_ref[...], wfc_ref[...], precision=_PREC) + bfc_ref[...]
    h = jnp.maximum(h, 0.0)
    y = jnp.dot(h, wl_ref[...], precision=_PREC).astype(jnp.bfloat16)
    y4_ref[0] = y[:, :HW]
    y4_ref[1] = y[:, HW:]
    z_ref[...] = jnp.dot(h, wr_ref[...], precision=_PREC) + b_ref[...]


def _tc_in(x, W_fc, b_fc, Wl, Wr, b):
    return pl.pallas_call(
        _tc_in_body,
        grid=(GRID,),
        in_specs=[
            pl.BlockSpec((BM, D_IN), lambda i: (i, 0)),
            pl.BlockSpec((D_IN, H), lambda i: (0, 0)),
            pl.BlockSpec((1, H), lambda i: (0, 0)),
            pl.BlockSpec((H, H), lambda i: (0, 0)),
            pl.BlockSpec((H, H), lambda i: (0, 0)),
            pl.BlockSpec((1, H), lambda i: (0, 0)),
        ],
        out_specs=[
            pl.BlockSpec((2, BM, HW), lambda i: (0, i, 0)),
            pl.BlockSpec((BM, H), lambda i: (i, 0)),
        ],
        out_shape=[
            jax.ShapeDtypeStruct((2, N, HW), jnp.bfloat16),
            jax.ShapeDtypeStruct((N, H), jnp.float32),
        ],
    )(x, W_fc, b_fc, Wl, Wr, b)


def _mid_body(l_ref, agg_ref, deg_in_ref, z_ref, deg_ref, g_ref, be_ref,
              wl_ref, wr_ref, b_ref, y4_ref, z2_ref, deg2_ref, traw_ref):
    lv = l_ref[0, 0]
    agg = jnp.concatenate(
        [agg_ref[0], agg_ref[1]], axis=1).astype(jnp.float32)
    deg_new = (deg_in_ref[0, :, 0:1] + deg_in_ref[1, :, 0:1]
               ).astype(jnp.float32)
    deg = jnp.where(lv == 1, deg_new, deg_ref[...])
    deg2_ref[...] = deg
    inv = 1.0 / jnp.maximum(deg, 1.0)
    traw = agg * inv + z_ref[...]
    traw_ref[...] = traw
    mu = jnp.mean(traw, axis=-1, keepdims=True)
    var = jnp.mean((traw - mu) * (traw - mu), axis=-1, keepdims=True)
    t = (traw - mu) * lax.rsqrt(var + 1e-5) * g_ref[...] + be_ref[...]
    t = jnp.maximum(t, 0.0)
    y = jnp.dot(t, wl_ref[...], precision=_PREC).astype(jnp.bfloat16)
    y4_ref[0] = y[:, :HW]
    y4_ref[1] = y[:, HW:]
    z2_ref[...] = jnp.dot(t, wr_ref[...], precision=_PREC) + b_ref[...]


def _tc_mid(lflag, aggM, degM, z, deg, g, be, Wl, Wr, b):
    return pl.pallas_call(
        _mid_body,
        grid=(GRID,),
        in_specs=[
            pl.BlockSpec((1, 1), lambda i: (0, 0)),
            pl.BlockSpec((2, BM, HW), lambda i: (0, i, 0)),
            pl.BlockSpec((2, BM, HW), lambda i: (0, i, 0)),
            pl.BlockSpec((BM, H), lambda i: (i, 0)),
            pl.BlockSpec((BM, 1), lambda i: (i, 0)),
            pl.BlockSpec((1, H), lambda i: (0, 0)),
            pl.BlockSpec((1, H), lambda i: (0, 0)),
            pl.BlockSpec((H, H), lambda i: (0, 0)),
            pl.BlockSpec((H, H), lambda i: (0, 0)),
            pl.BlockSpec((1, H), lambda i: (0, 0)),
        ],
        out_specs=[
            pl.BlockSpec((2, BM, HW), lambda i: (0, i, 0)),
            pl.BlockSpec((BM, H), lambda i: (i, 0)),
            pl.BlockSpec((BM, 1), lambda i: (i, 0)),
            pl.BlockSpec((BM, H), lambda i: (i, 0)),
        ],
        out_shape=[
            jax.ShapeDtypeStruct((2, N, HW), jnp.bfloat16),
            jax.ShapeDtypeStruct((N, H), jnp.float32),
            jax.ShapeDtypeStruct((N, 1), jnp.float32),
            jax.ShapeDtypeStruct((N, H), jnp.float32),
        ],
    )(lflag, aggM, degM, z, deg, g, be, Wl, Wr, b)


def _pool_body(ne_ref, batch_ref, ge_ref):
    i = pl.program_id(0)
    node = ne_ref[...]
    b = batch_ref[0, 0]
    p = (b[:, None] == lax.broadcasted_iota(jnp.int32, (BM, G), 1)
         ).astype(jnp.float32)
    contrib = lax.dot_general(p, node, (((0,), (0,)), ((), ())),
                              precision=_PREC)

    @pl.when(i == 0)
    def _():
        ge_ref[...] = jnp.zeros_like(ge_ref)

    ge_ref[...] += contrib


def _tc_pool(node_embed, batch2):
    return pl.pallas_call(
        _pool_body,
        grid=(GRID,),
        in_specs=[
            pl.BlockSpec((BM, OUT), lambda i: (i, 0)),
            pl.BlockSpec((1, 1, BM), lambda i: (i, 0, 0)),
        ],
        out_specs=pl.BlockSpec((G, OUT), lambda i: (0, 0)),
        out_shape=jax.ShapeDtypeStruct((G, OUT), jnp.float32),
    )(node_embed, batch2)


def kernel(x, edge_index, batch, W_fc, b_fc,
           Wl1, bl1, Wr1, br1, g1, be1,
           Wl2, bl2, Wr2, br2, g2, be2,
           Wl3, bl3, Wr3, br3):
    src = edge_index[0]
    dst = edge_index[1]
    src_p = jnp.concatenate(
        [src, jnp.zeros((E_PAD - E,), jnp.int32)]).reshape(NCHUNK, CHUNK)
    dst_p = jnp.concatenate(
        [dst, jnp.full((E_PAD - E,), DUMMY, jnp.int32)]).reshape(NCHUNK, CHUNK)
    batch2 = batch.reshape(GRID, 1, BM)

    y1, z1 = _tc_in(x, W_fc, b_fc.reshape(1, H), Wl1, Wr1,
                    (bl1 + br1).reshape(1, H))

    ones64 = jnp.ones((1, H), jnp.float32)
    zeros64 = jnp.zeros((1, H), jnp.float32)
    zerosW = jnp.zeros((H, H), jnp.float32)
    xs = (
        jnp.array([[[1]], [[2]], [[3]]], jnp.int32),
        jnp.stack([jnp.ones((16,), jnp.int32)] +
                  [jnp.zeros((16,), jnp.int32)] * 2),
        jnp.stack([g1.reshape(1, H), g2.reshape(1, H), ones64]),
        jnp.stack([be1.reshape(1, H), be2.reshape(1, H), zeros64]),
        jnp.stack([Wl2, Wl3, zerosW]),
        jnp.stack([Wr2, Wr3, zerosW]),
        jnp.stack([(bl2 + br2).reshape(1, H),
                   (bl3 + br3).reshape(1, H), zeros64]),
    )

    def body(carry, xts):
        y4, z, deg, _ = carry
        lflag, scflag, g, be, Wl, Wr, b = xts
        aggM, degM = _sc_agg(scflag, y4, src_p, dst_p)
        y4n, zn, degn, traw = _tc_mid(lflag, aggM, degM, z, deg, g, be,
                                      Wl, Wr, b)
        return (y4n, zn, degn, traw), None

    init = (y1, z1, jnp.zeros((N, 1), jnp.float32),
            jnp.zeros((N, H), jnp.float32))
    (_, _, _, node_embed), _ = lax.scan(body, init, xs)
    graph_embed = _tc_pool(node_embed, batch2)
    return node_embed, graph_embed
